# Initial kernel scaffold; baseline (speedup 1.0000x reference)
import jax
import jax.numpy as jnp
from jax.experimental import pallas as pl


def kernel(constraint_features, edge_indices, edge_features, variable_features, params):
    raise NotImplementedError("write your pallas kernel here")


# speedup vs baseline: 1.0231x; 1.0231x over previous
import functools

import jax
import jax.numpy as jnp
from jax import lax
from jax.experimental import pallas as pl
from jax.experimental.pallas import tpu as pltpu
from jax.experimental.pallas import tpu_sc as plsc

N_NODES = 50000
N_EDGES = 800000
EMB = 64
HALF = 32
EB = 128
NBLK = N_EDGES // EB
N_Q = 6256
NSLICE = 8
QROWS = 6272
ZROWS = 392
EPS = 1e-5

_MESH = plsc.VectorSubcoreMesh(core_axis_name="c", subcore_axis_name="s")


@functools.partial(
    pl.kernel,
    out_type=jax.ShapeDtypeStruct((N_EDGES, 2 * EMB), jnp.float32),
    mesh=_MESH,
    scratch_types=[
        pltpu.VMEM((EB,), jnp.int32),
        pltpu.VMEM((EB,), jnp.int32),
        pltpu.VMEM((EB, 2 * EMB), jnp.float32),
        pltpu.VMEM((EB, 2 * EMB), jnp.float32),
        pltpu.SemaphoreType.DMA,
        pltpu.SemaphoreType.DMA,
    ],
)
def _sc_gather(t_hbm, ai_hbm, bi_hbm, tab_hbm,
               idxa_v, idxb_v, bufd_v, bufs_v, sema, semb):
    c = lax.axis_index("c")
    s = lax.axis_index("s")
    wid = s * 2 + c

    def step(i, carry):
        bid = i * 32 + wid

        @pl.when(bid < NBLK)
        def _():
            off = bid * EB
            pltpu.sync_copy(ai_hbm.at[pl.ds(off, EB)], idxa_v)
            pltpu.sync_copy(bi_hbm.at[pl.ds(off, EB)], idxb_v)
            cpd = pltpu.async_copy(t_hbm.at[idxa_v], bufd_v, sema)
            cps = pltpu.async_copy(t_hbm.at[idxb_v], bufs_v, semb)
            cpd.wait()
            cps.wait()

            def mrow(r, carry2):
                bufd_v[r, pl.ds(EMB, 16)] = bufs_v[r, pl.ds(EMB, 16)]
                bufd_v[r, pl.ds(EMB + 16, 16)] = bufs_v[r, pl.ds(EMB + 16, 16)]
                bufd_v[r, pl.ds(EMB + 32, 16)] = bufs_v[r, pl.ds(EMB + 32, 16)]
                bufd_v[r, pl.ds(EMB + 48, 16)] = bufs_v[r, pl.ds(EMB + 48, 16)]
                return carry2
            lax.fori_loop(0, EB, mrow, 0)
            pltpu.sync_copy(bufd_v, tab_hbm.at[pl.ds(off, EB), :])
        return carry
    lax.fori_loop(0, (NBLK + 31) // 32, step, 0)


@functools.partial(
    pl.kernel,
    out_type=jax.ShapeDtypeStruct((NSLICE * QROWS, 2 * EMB), jnp.float32),
    mesh=_MESH,
    scratch_types=[
        pltpu.VMEM((EB,), jnp.int32),
        pltpu.VMEM((EB,), jnp.int32),
        pltpu.VMEM((EB, 2 * EMB), jnp.float32),
        pltpu.VMEM((ZROWS, 2 * EMB), jnp.float32),
        pltpu.VMEM_SHARED((QROWS, 2 * EMB), jnp.float32),
    ],
)
def _sc_scatter_add(u_hbm, di_hbm, out_hbm,
                    idx_v, idxq_v, u_v, zed_v, acc_sh):
    c = lax.axis_index("c")
    s = lax.axis_index("s")
    zrow = jnp.zeros((16,), jnp.float32)

    def fzero(i, carry):
        for k in range(8):
            zed_v[i, pl.ds(16 * k, 16)] = zrow
        return carry
    lax.fori_loop(0, ZROWS, fzero, 0)

    for p in range(NSLICE // 2):
        q = c * (NSLICE // 2) + p
        base = s * ZROWS
        pltpu.sync_copy(zed_v, acc_sh.at[pl.ds(base, ZROWS), :])
        plsc.subcore_barrier()

        def step(i, carry):
            bid = i * 16 + s

            @pl.when(bid < NBLK)
            def _():
                off = bid * EB
                pltpu.sync_copy(di_hbm.at[pl.ds(off, EB)], idx_v)
                pltpu.sync_copy(u_hbm.at[pl.ds(off, EB), :], u_v)
                junk = N_Q + s
                for k in range(EB // 16):
                    iv = idx_v[pl.ds(16 * k, 16)] - q * N_Q
                    ok = (iv >= 0) & (iv < N_Q)
                    idxq_v[pl.ds(16 * k, 16)] = jnp.where(ok, iv, junk)
                pltpu.sync_copy(u_v, acc_sh.at[idxq_v], add=True)
            return carry
        lax.fori_loop(0, (NBLK + 15) // 16, step, 0)
        plsc.subcore_barrier()

        rows = pl.ds(s * ZROWS, ZROWS)
        out_rows = pl.ds(q * QROWS + s * ZROWS, ZROWS)
        pltpu.sync_copy(acc_sh.at[rows, :], out_hbm.at[out_rows, :])
        plsc.subcore_barrier()


def _ln_rows(x, g, b):
    m = jnp.mean(x, axis=-1, keepdims=True)
    v = jnp.mean((x - m) * (x - m), axis=-1, keepdims=True)
    return (x - m) * lax.rsqrt(v + EPS) * g + b


def _tc_embed_body(cons_ref, var_ref,
                   clng_ref, clnb_ref, cw1_ref, cb1_ref, cw2_ref, cb2_ref,
                   vlng_ref, vlnb_ref, vw1_ref, vb1_ref, vw2_ref, vb2_ref,
                   wl1_ref, ab1_ref, wr1_ref, wl2_ref, ab2_ref,
                   consemb_ref, varemb_ref, t1_ref, a2_ref):
    cons = _ln_rows(cons_ref[...], clng_ref[...], clnb_ref[...])
    cons = jnp.maximum(jnp.dot(cons, cw1_ref[...],
                               preferred_element_type=jnp.float32,
                 precision=lax.Precision.HIGHEST)
                       + cb1_ref[...], 0.0)
    cons = jnp.maximum(jnp.dot(cons, cw2_ref[...],
                               preferred_element_type=jnp.float32,
                 precision=lax.Precision.HIGHEST)
                       + cb2_ref[...], 0.0)
    consemb_ref[...] = cons
    a1 = jnp.dot(cons, wl1_ref[...],
                 preferred_element_type=jnp.float32,
                 precision=lax.Precision.HIGHEST) + ab1_ref[...]

    var = _ln_rows(var_ref[...], vlng_ref[...], vlnb_ref[...])
    var = jnp.maximum(jnp.dot(var, vw1_ref[...],
                              preferred_element_type=jnp.float32,
                 precision=lax.Precision.HIGHEST)
                      + vb1_ref[...], 0.0)
    var = jnp.maximum(jnp.dot(var, vw2_ref[...],
                              preferred_element_type=jnp.float32,
                 precision=lax.Precision.HIGHEST)
                      + vb2_ref[...], 0.0)
    varemb_ref[...] = var
    b1 = jnp.dot(var, wr1_ref[...],
                 preferred_element_type=jnp.float32,
                 precision=lax.Precision.HIGHEST)
    t1_ref[...] = jnp.concatenate([a1, b1], axis=-1)
    a2_ref[...] = jnp.dot(var, wl2_ref[...],
                          preferred_element_type=jnp.float32,
                 precision=lax.Precision.HIGHEST) + ab2_ref[...]


def _tc_edge_ln_body(tab_ref, g_ref, b_ref, onehot_ref, u_ref):
    tab = tab_ref[...]
    t = tab[:, :EMB] + tab[:, EMB:]
    u = jnp.maximum(_ln_rows(t, g_ref[...], b_ref[...]), 0.0)
    tag = jnp.broadcast_to(onehot_ref[...], u.shape)
    u_ref[...] = jnp.concatenate([u, tag], axis=-1)


def _tc_post_node(ag_ref, right_ref,
                  wf_ref, bf_ref, pcg_ref, pcb_ref,
                  wo1_ref, bo1_ref, wo2_ref, bo2_ref):
    ag = ag_ref[...]
    aggw = (jnp.dot(ag[:, :EMB], wf_ref[...],
                    preferred_element_type=jnp.float32,
                 precision=lax.Precision.HIGHEST)
            + ag[:, EMB:EMB + 1] * bf_ref[...])
    h1 = _ln_rows(aggw, pcg_ref[...], pcb_ref[...])
    wo1 = wo1_ref[...]
    h = jnp.maximum(jnp.dot(h1, wo1[:EMB, :],
                            preferred_element_type=jnp.float32,
                 precision=lax.Precision.HIGHEST)
                    + jnp.dot(right_ref[...], wo1[EMB:, :],
                              preferred_element_type=jnp.float32,
                 precision=lax.Precision.HIGHEST)
                    + bo1_ref[...], 0.0)
    return jnp.dot(h, wo2_ref[...],
                   preferred_element_type=jnp.float32,
                 precision=lax.Precision.HIGHEST) + bo2_ref[...]


def _tc_post1_body(ag_ref, right_ref,
                   wf_ref, bf_ref, pcg_ref, pcb_ref,
                   wo1_ref, bo1_ref, wo2_ref, bo2_ref,
                   a2_ref, wr2_ref, t2_ref):
    node = _tc_post_node(ag_ref, right_ref,
                         wf_ref, bf_ref, pcg_ref, pcb_ref,
                         wo1_ref, bo1_ref, wo2_ref, bo2_ref)
    b2 = jnp.dot(node, wr2_ref[...], preferred_element_type=jnp.float32,
                 precision=lax.Precision.HIGHEST)
    t2_ref[...] = jnp.concatenate([a2_ref[...], b2], axis=-1)


def _tc_post2_body(ag_ref, right_ref,
                   wf_ref, bf_ref, pcg_ref, pcb_ref,
                   wo1_ref, bo1_ref, wo2_ref, bo2_ref,
                   w1_ref, b1_ref, w2_ref, out_ref):
    node = _tc_post_node(ag_ref, right_ref,
                         wf_ref, bf_ref, pcg_ref, pcb_ref,
                         wo1_ref, bo1_ref, wo2_ref, bo2_ref)
    h = jnp.maximum(jnp.dot(node, w1_ref[...],
                            preferred_element_type=jnp.float32,
                 precision=lax.Precision.HIGHEST)
                    + b1_ref[...], 0.0)
    out_ref[...] = jnp.dot(h, w2_ref[...],
                           preferred_element_type=jnp.float32,
                 precision=lax.Precision.HIGHEST)


def _full_spec(shape):
    return pl.BlockSpec(shape, lambda i: (0,) * len(shape))


def _row_spec(blk, shape):
    return pl.BlockSpec((blk,) + shape[1:],
                        lambda i: (i,) + (0,) * (len(shape) - 1))


def _tc_call(body, grid, n_rows, blk, ins, row_mask, out_shapes):
    in_specs = [_row_spec(blk, x.shape) if is_row else _full_spec(x.shape)
                for x, is_row in zip(ins, row_mask)]
    out_specs = [_row_spec(blk, s) for s in out_shapes]
    return pl.pallas_call(
        body,
        grid=(grid,),
        in_specs=in_specs,
        out_specs=out_specs,
        out_shape=[jax.ShapeDtypeStruct(s, jnp.float32) for s in out_shapes],
    )(*ins)


def _conv_edge_phase(t, ai, bi, fg, fb):
    tab = _sc_gather(t, ai, bi)
    blk = 4000
    onehot = jnp.where(jnp.arange(EMB) == 0, 1.0, 0.0
                       ).astype(jnp.float32).reshape(1, EMB)
    return _tc_call(
        _tc_edge_ln_body, N_EDGES // blk, N_EDGES, blk,
        [tab, fg, fb, onehot], [True, False, False, False],
        [(N_EDGES, 2 * EMB)])[0]


def _r2(x):
    return x.reshape(1, -1)


def kernel(constraint_features, edge_indices, edge_features, variable_features, params):
    del edge_features
    p = params
    cidx = edge_indices[0]
    vidx = edge_indices[1]
    vc, cv = p['vc'], p['cv']
    econst = p['edge_ln_b'][0]
    ab1 = _r2(vc['bl'] + econst * vc['We'][0])
    ab2 = _r2(cv['bl'] + econst * cv['We'][0])

    blk_n = 2000
    grid_n = N_NODES // blk_n
    consemb, varemb, t1, a2 = _tc_call(
        _tc_embed_body, grid_n, N_NODES, blk_n,
        [constraint_features, variable_features,
         _r2(p['cons_ln_g']), _r2(p['cons_ln_b']),
         p['cons_W1'], _r2(p['cons_b1']), p['cons_W2'], _r2(p['cons_b2']),
         _r2(p['var_ln_g']), _r2(p['var_ln_b']),
         p['var_W1'], _r2(p['var_b1']), p['var_W2'], _r2(p['var_b2']),
         vc['Wl'], ab1, vc['Wr'], cv['Wl'], ab2],
        [True, True] + [False] * 17,
        [(N_NODES, EMB), (N_NODES, EMB), (N_NODES, 2 * EMB), (N_NODES, EMB)])

    u = _conv_edge_phase(t1, cidx, vidx,
                         _r2(vc['fln_g']), _r2(vc['fln_b']))
    ag = _assemble_agg(_sc_scatter_add(u, cidx))
    t2 = _tc_call(
        _tc_post1_body, grid_n, N_NODES, blk_n,
        [ag, consemb,
         vc['Wf'], _r2(vc['bf']), _r2(vc['pc_g']), _r2(vc['pc_b']),
         vc['Wo1'], _r2(vc['bo1']), vc['Wo2'], _r2(vc['bo2']),
         a2, cv['Wr']],
        [True, True] + [False] * 8 + [True, False],
        [(N_NODES, 2 * EMB)])[0]

    u = _conv_edge_phase(t2, vidx, cidx,
                         _r2(cv['fln_g']), _r2(cv['fln_b']))
    ag = _assemble_agg(_sc_scatter_add(u, vidx))
    out = _tc_call(
        _tc_post2_body, grid_n, N_NODES, blk_n,
        [ag, varemb,
         cv['Wf'], _r2(cv['bf']), _r2(cv['pc_g']), _r2(cv['pc_b']),
         cv['Wo1'], _r2(cv['bo1']), cv['Wo2'], _r2(cv['bo2']),
         p['out_W1'], _r2(p['out_b1']), p['out_W2']],
        [True, True] + [False] * 11,
        [(N_NODES, 1)])[0]
    return jnp.squeeze(out, -1)


def _assemble_agg(res):
    q = res.reshape(NSLICE, QROWS, 2 * EMB)[:, :N_Q, :]
    return q.reshape(NSLICE * N_Q, 2 * EMB)[:N_NODES]


# speedup vs baseline: 1.5968x; 1.5607x over previous
import functools

import jax
import jax.numpy as jnp
from jax import lax
from jax.experimental import pallas as pl
from jax.experimental.pallas import tpu as pltpu
from jax.experimental.pallas import tpu_sc as plsc

N_NODES = 50000
N_EDGES = 800000
EMB = 64
HALF = 32
EB = 128
NBLK = N_EDGES // EB
N_Q = 8352
NSLICE = 6
QROWS = 8448
SUBROWS = 528
ZROWS = 48
EPS = 1e-5

_MESH = plsc.VectorSubcoreMesh(core_axis_name="c", subcore_axis_name="s")


@functools.partial(
    pl.kernel,
    out_type=jax.ShapeDtypeStruct((N_EDGES, 2 * EMB), jnp.float32),
    mesh=_MESH,
    scratch_types=[
        pltpu.VMEM((EB,), jnp.int32),
        pltpu.VMEM((EB,), jnp.int32),
        pltpu.VMEM((EB, 2 * EMB), jnp.float32),
        pltpu.VMEM((EB, 2 * EMB), jnp.float32),
        pltpu.SemaphoreType.DMA,
        pltpu.SemaphoreType.DMA,
    ],
)
def _sc_gather(t_hbm, ai_hbm, bi_hbm, tab_hbm,
               idxa_v, idxb_v, bufd_v, bufs_v, sema, semb):
    c = lax.axis_index("c")
    s = lax.axis_index("s")
    wid = s * 2 + c

    def step(i, carry):
        bid = i * 32 + wid

        @pl.when(bid < NBLK)
        def _():
            off = bid * EB
            pltpu.sync_copy(ai_hbm.at[pl.ds(off, EB)], idxa_v)
            pltpu.sync_copy(bi_hbm.at[pl.ds(off, EB)], idxb_v)
            cpd = pltpu.async_copy(t_hbm.at[idxa_v], bufd_v, sema)
            cps = pltpu.async_copy(t_hbm.at[idxb_v], bufs_v, semb)
            cpd.wait()
            cps.wait()

            def mrow(r, carry2):
                bufd_v[r, pl.ds(EMB, 16)] = bufs_v[r, pl.ds(EMB, 16)]
                bufd_v[r, pl.ds(EMB + 16, 16)] = bufs_v[r, pl.ds(EMB + 16, 16)]
                bufd_v[r, pl.ds(EMB + 32, 16)] = bufs_v[r, pl.ds(EMB + 32, 16)]
                bufd_v[r, pl.ds(EMB + 48, 16)] = bufs_v[r, pl.ds(EMB + 48, 16)]
                return carry2
            lax.fori_loop(0, EB, mrow, 0)
            pltpu.sync_copy(bufd_v, tab_hbm.at[pl.ds(off, EB), :])
        return carry
    lax.fori_loop(0, (NBLK + 31) // 32, step, 0)


@functools.partial(
    pl.kernel,
    out_type=jax.ShapeDtypeStruct((NSLICE * QROWS, 2 * EMB), jnp.float32),
    mesh=_MESH,
    scratch_types=[
        pltpu.VMEM((2, EB), jnp.int32),
        pltpu.VMEM((2, EB), jnp.int32),
        pltpu.VMEM((2, EB, 2 * EMB), jnp.float32),
        pltpu.VMEM((ZROWS, 2 * EMB), jnp.float32),
        pltpu.VMEM_SHARED((QROWS, 2 * EMB), jnp.float32),
        pltpu.SemaphoreType.DMA,
        pltpu.SemaphoreType.DMA,
        pltpu.SemaphoreType.DMA,
        pltpu.SemaphoreType.DMA,
    ],
)
def _sc_scatter_add(u_hbm, di_hbm, out_hbm,
                    idx_v, idxq_v, u_v, zed_v, acc_sh,
                    lsem0, lsem1, ssem0, ssem1):
    c = lax.axis_index("c")
    s = lax.axis_index("s")
    zrow = jnp.zeros((16,), jnp.float32)

    def fzero(i, carry):
        for k in range(8):
            zed_v[i, pl.ds(16 * k, 16)] = zrow
        return carry
    lax.fori_loop(0, ZROWS, fzero, 0)

    for p in range(NSLICE // 2):
        q = c * (NSLICE // 2) + p
        for j in range(SUBROWS // ZROWS):
            pltpu.sync_copy(zed_v,
                            acc_sh.at[pl.ds(s * SUBROWS + j * ZROWS, ZROWS), :])
        plsc.subcore_barrier()

        lsems = (lsem0, lsem1)
        ssems = (ssem0, ssem1)

        def step(i, carry):
            for k in range(2):
                bid = (2 * i + k) * 16 + s

                @pl.when(bid < NBLK)
                def _():
                    off = bid * EB
                    pltpu.async_copy(di_hbm.at[pl.ds(off, EB)],
                                     idx_v.at[k], lsems[k])
                    pltpu.async_copy(u_hbm.at[pl.ds(off, EB), :],
                                     u_v.at[k], lsems[k])
            for k in range(2):
                bid = (2 * i + k) * 16 + s

                @pl.when(bid < NBLK)
                def _():
                    off = bid * EB
                    pltpu.make_async_copy(di_hbm.at[pl.ds(off, EB)],
                                          idx_v.at[k], lsems[k]).wait()
                    pltpu.make_async_copy(u_hbm.at[pl.ds(off, EB), :],
                                          u_v.at[k], lsems[k]).wait()
                    junk = N_Q + s
                    for kk in range(EB // 16):
                        iv = idx_v[k, pl.ds(16 * kk, 16)] - q * N_Q
                        ok = (iv >= 0) & (iv < N_Q)
                        idxq_v[k, pl.ds(16 * kk, 16)] = jnp.where(ok, iv, junk)
                    pltpu.async_copy(u_v.at[k], acc_sh.at[idxq_v.at[k]],
                                     ssems[k], add=True)
            for k in range(2):
                bid = (2 * i + k) * 16 + s

                @pl.when(bid < NBLK)
                def _():
                    pltpu.make_async_copy(u_v.at[k], acc_sh.at[idxq_v.at[k]],
                                          ssems[k]).wait()
            return carry
        lax.fori_loop(0, (NBLK + 31) // 32, step, 0)
        plsc.subcore_barrier()

        rows = pl.ds(s * SUBROWS, SUBROWS)
        out_rows = pl.ds(q * QROWS + s * SUBROWS, SUBROWS)
        pltpu.sync_copy(acc_sh.at[rows, :], out_hbm.at[out_rows, :])
        plsc.subcore_barrier()


def _ln_rows(x, g, b):
    m = jnp.mean(x, axis=-1, keepdims=True)
    v = jnp.mean((x - m) * (x - m), axis=-1, keepdims=True)
    return (x - m) / jnp.sqrt(v + EPS) * g + b


def _tc_embed_body(cons_ref, var_ref,
                   clng_ref, clnb_ref, cw1_ref, cb1_ref, cw2_ref, cb2_ref,
                   vlng_ref, vlnb_ref, vw1_ref, vb1_ref, vw2_ref, vb2_ref,
                   wl1_ref, ab1_ref, wr1_ref, wl2_ref, ab2_ref,
                   consemb_ref, varemb_ref, t1_ref, a2_ref):
    cons = _ln_rows(cons_ref[...], clng_ref[...], clnb_ref[...])
    cons = jnp.maximum(jnp.dot(cons, cw1_ref[...],
                               preferred_element_type=jnp.float32)
                       + cb1_ref[...], 0.0)
    cons = jnp.maximum(jnp.dot(cons, cw2_ref[...],
                               preferred_element_type=jnp.float32)
                       + cb2_ref[...], 0.0)
    consemb_ref[...] = cons
    a1 = jnp.dot(cons, wl1_ref[...],
                 preferred_element_type=jnp.float32) + ab1_ref[...]

    var = _ln_rows(var_ref[...], vlng_ref[...], vlnb_ref[...])
    var = jnp.maximum(jnp.dot(var, vw1_ref[...],
                              preferred_element_type=jnp.float32)
                      + vb1_ref[...], 0.0)
    var = jnp.maximum(jnp.dot(var, vw2_ref[...],
                              preferred_element_type=jnp.float32)
                      + vb2_ref[...], 0.0)
    varemb_ref[...] = var
    b1 = jnp.dot(var, wr1_ref[...],
                 preferred_element_type=jnp.float32)
    t1_ref[...] = jnp.concatenate([a1, b1], axis=-1)
    a2_ref[...] = jnp.dot(var, wl2_ref[...],
                          preferred_element_type=jnp.float32) + ab2_ref[...]


def _tc_edge_ln_body(tab_ref, g_ref, b_ref, wf_ref, bf_ref, zero_ref, u_ref):
    tab = tab_ref[...]
    t = tab[:, :EMB] + tab[:, EMB:]
    u = jnp.maximum(_ln_rows(t, g_ref[...], b_ref[...]), 0.0)
    msg = jnp.dot(u, wf_ref[...], preferred_element_type=jnp.float32) + bf_ref[...]
    pad = jnp.broadcast_to(zero_ref[...], msg.shape)
    u_ref[...] = jnp.concatenate([msg, pad], axis=-1)


def _tc_post_node(ag_ref, right_ref,
                  pcg_ref, pcb_ref,
                  wo1_ref, bo1_ref, wo2_ref, bo2_ref):
    h1 = _ln_rows(ag_ref[...][:, :EMB], pcg_ref[...], pcb_ref[...])
    cat = jnp.concatenate([h1, right_ref[...]], axis=-1)
    h = jnp.maximum(jnp.dot(cat, wo1_ref[...],
                            preferred_element_type=jnp.float32)
                    + bo1_ref[...], 0.0)
    return jnp.dot(h, wo2_ref[...],
                   preferred_element_type=jnp.float32) + bo2_ref[...]


def _tc_post1_body(ag_ref, right_ref,
                   pcg_ref, pcb_ref,
                   wo1_ref, bo1_ref, wo2_ref, bo2_ref,
                   a2_ref, wr2_ref, t2_ref):
    node = _tc_post_node(ag_ref, right_ref,
                         pcg_ref, pcb_ref,
                         wo1_ref, bo1_ref, wo2_ref, bo2_ref)
    b2 = jnp.dot(node, wr2_ref[...], preferred_element_type=jnp.float32)
    t2_ref[...] = jnp.concatenate([a2_ref[...], b2], axis=-1)


def _tc_post2_body(ag_ref, right_ref,
                   pcg_ref, pcb_ref,
                   wo1_ref, bo1_ref, wo2_ref, bo2_ref,
                   w1_ref, b1_ref, w2_ref, out_ref):
    node = _tc_post_node(ag_ref, right_ref,
                         pcg_ref, pcb_ref,
                         wo1_ref, bo1_ref, wo2_ref, bo2_ref)
    h = jnp.maximum(jnp.dot(node, w1_ref[...],
                            preferred_element_type=jnp.float32)
                    + b1_ref[...], 0.0)
    out_ref[...] = jnp.dot(h, w2_ref[...],
                           preferred_element_type=jnp.float32)


def _full_spec(shape):
    return pl.BlockSpec(shape, lambda i: (0,) * len(shape))


def _row_spec(blk, shape):
    return pl.BlockSpec((blk,) + shape[1:],
                        lambda i: (i,) + (0,) * (len(shape) - 1))


def _tc_call(body, grid, n_rows, blk, ins, row_mask, out_shapes):
    in_specs = [_row_spec(blk, x.shape) if is_row else _full_spec(x.shape)
                for x, is_row in zip(ins, row_mask)]
    out_specs = [_row_spec(blk, s) for s in out_shapes]
    return pl.pallas_call(
        body,
        grid=(grid,),
        in_specs=in_specs,
        out_specs=out_specs,
        out_shape=[jax.ShapeDtypeStruct(s, jnp.float32) for s in out_shapes],
    )(*ins)


def _conv_edge_phase(t, ai, bi, fg, fb, wf, bf):
    tab = _sc_gather(t, ai, bi)
    blk = 4000
    zero = jnp.zeros((1, EMB), jnp.float32)
    return _tc_call(
        _tc_edge_ln_body, N_EDGES // blk, N_EDGES, blk,
        [tab, fg, fb, wf, bf, zero], [True] + [False] * 5,
        [(N_EDGES, 2 * EMB)])[0]


def _r2(x):
    return x.reshape(1, -1)


def kernel(constraint_features, edge_indices, edge_features, variable_features, params):
    del edge_features
    p = params
    cidx = edge_indices[0]
    vidx = edge_indices[1]
    vc, cv = p['vc'], p['cv']
    econst = p['edge_ln_b'][0]
    ab1 = _r2(vc['bl'] + econst * vc['We'][0])
    ab2 = _r2(cv['bl'] + econst * cv['We'][0])

    blk_n = 2000
    grid_n = N_NODES // blk_n
    consemb, varemb, t1, a2 = _tc_call(
        _tc_embed_body, grid_n, N_NODES, blk_n,
        [constraint_features, variable_features,
         _r2(p['cons_ln_g']), _r2(p['cons_ln_b']),
         p['cons_W1'], _r2(p['cons_b1']), p['cons_W2'], _r2(p['cons_b2']),
         _r2(p['var_ln_g']), _r2(p['var_ln_b']),
         p['var_W1'], _r2(p['var_b1']), p['var_W2'], _r2(p['var_b2']),
         vc['Wl'], ab1, vc['Wr'], cv['Wl'], ab2],
        [True, True] + [False] * 17,
        [(N_NODES, EMB), (N_NODES, EMB), (N_NODES, 2 * EMB), (N_NODES, EMB)])

    u = _conv_edge_phase(t1, cidx, vidx,
                         _r2(vc['fln_g']), _r2(vc['fln_b']),
                         vc['Wf'], _r2(vc['bf']))
    ag = _assemble_agg(_sc_scatter_add(u, cidx))
    t2 = _tc_call(
        _tc_post1_body, grid_n, N_NODES, blk_n,
        [ag, consemb,
         _r2(vc['pc_g']), _r2(vc['pc_b']),
         vc['Wo1'], _r2(vc['bo1']), vc['Wo2'], _r2(vc['bo2']),
         a2, cv['Wr']],
        [True, True] + [False] * 6 + [True, False],
        [(N_NODES, 2 * EMB)])[0]

    u = _conv_edge_phase(t2, vidx, cidx,
                         _r2(cv['fln_g']), _r2(cv['fln_b']),
                         cv['Wf'], _r2(cv['bf']))
    ag = _assemble_agg(_sc_scatter_add(u, vidx))
    out = _tc_call(
        _tc_post2_body, grid_n, N_NODES, blk_n,
        [ag, varemb,
         _r2(cv['pc_g']), _r2(cv['pc_b']),
         cv['Wo1'], _r2(cv['bo1']), cv['Wo2'], _r2(cv['bo2']),
         p['out_W1'], _r2(p['out_b1']), p['out_W2']],
        [True, True] + [False] * 9,
        [(N_NODES, 1)])[0]
    return jnp.squeeze(out, -1)


def _assemble_agg(res):
    q = res.reshape(NSLICE, QROWS, 2 * EMB)[:, :N_Q, :]
    return q.reshape(NSLICE * N_Q, 2 * EMB)[:N_NODES]


# speedup vs baseline: 1.7274x; 1.0818x over previous
import functools

import jax
import jax.numpy as jnp
from jax import lax
from jax.experimental import pallas as pl
from jax.experimental.pallas import tpu as pltpu
from jax.experimental.pallas import tpu_sc as plsc

N_NODES = 50000
N_EDGES = 800000
EMB = 64
HALF = 32
EB = 128
NBLK = N_EDGES // EB
N_Q = 8352
NSLICE = 6
QROWS = 8448
SUBROWS = 528
ZROWS = 48
EPS = 1e-5

_MESH = plsc.VectorSubcoreMesh(core_axis_name="c", subcore_axis_name="s")


@functools.partial(
    pl.kernel,
    out_type=jax.ShapeDtypeStruct((N_EDGES, 2 * EMB), jnp.float32),
    mesh=_MESH,
    scratch_types=[
        pltpu.VMEM((2, EB), jnp.int32),
        pltpu.VMEM((2, EB), jnp.int32),
        pltpu.VMEM((2, EB, 2 * EMB), jnp.float32),
        pltpu.VMEM((2, EB, 2 * EMB), jnp.float32),
        pltpu.SemaphoreType.DMA,
        pltpu.SemaphoreType.DMA,
        pltpu.SemaphoreType.DMA,
        pltpu.SemaphoreType.DMA,
    ],
)
def _sc_gather(t_hbm, ai_hbm, bi_hbm, tab_hbm,
               idxa_v, idxb_v, bufd_v, bufs_v, sem0, sem1, wsem0, wsem1):
    c = lax.axis_index("c")
    s = lax.axis_index("s")
    wid = s * 2 + c

    sems = (sem0, sem1)
    wsems = (wsem0, wsem1)

    def step(i, carry):
        for k in range(2):
            bid = (2 * i + k) * 32 + wid

            @pl.when(bid < NBLK)
            def _():
                off = bid * EB
                pltpu.async_copy(ai_hbm.at[pl.ds(off, EB)],
                                 idxa_v.at[k], sems[k])
                pltpu.async_copy(bi_hbm.at[pl.ds(off, EB)],
                                 idxb_v.at[k], sems[k])
        for k in range(2):
            bid = (2 * i + k) * 32 + wid

            @pl.when(bid < NBLK)
            def _():
                off = bid * EB
                pltpu.make_async_copy(ai_hbm.at[pl.ds(off, EB)],
                                      idxa_v.at[k], sems[k]).wait()
                pltpu.make_async_copy(bi_hbm.at[pl.ds(off, EB)],
                                      idxb_v.at[k], sems[k]).wait()
                pltpu.async_copy(t_hbm.at[idxa_v.at[k]], bufd_v.at[k], sems[k])
                pltpu.async_copy(t_hbm.at[idxb_v.at[k]], bufs_v.at[k], sems[k])
        for k in range(2):
            bid = (2 * i + k) * 32 + wid

            @pl.when(bid < NBLK)
            def _():
                off = bid * EB
                pltpu.make_async_copy(t_hbm.at[idxa_v.at[k]],
                                      bufd_v.at[k], sems[k]).wait()
                pltpu.make_async_copy(t_hbm.at[idxb_v.at[k]],
                                      bufs_v.at[k], sems[k]).wait()

                def mrow(r, carry2):
                    for kk in range(4):
                        bufd_v[k, r, pl.ds(EMB + 16 * kk, 16)] = (
                            bufs_v[k, r, pl.ds(EMB + 16 * kk, 16)])
                    return carry2
                lax.fori_loop(0, EB, mrow, 0)
                pltpu.async_copy(bufd_v.at[k],
                                 tab_hbm.at[pl.ds(off, EB), :], wsems[k])
        for k in range(2):
            bid = (2 * i + k) * 32 + wid

            @pl.when(bid < NBLK)
            def _():
                off = bid * EB
                pltpu.make_async_copy(bufd_v.at[k],
                                      tab_hbm.at[pl.ds(off, EB), :],
                                      wsems[k]).wait()
        return carry
    lax.fori_loop(0, (NBLK + 63) // 64, step, 0)


@functools.partial(
    pl.kernel,
    out_type=jax.ShapeDtypeStruct((NSLICE * QROWS, 2 * EMB), jnp.float32),
    mesh=_MESH,
    scratch_types=[
        pltpu.VMEM((2, EB), jnp.int32),
        pltpu.VMEM((2, EB), jnp.int32),
        pltpu.VMEM((2, EB, 2 * EMB), jnp.float32),
        pltpu.VMEM((ZROWS, 2 * EMB), jnp.float32),
        pltpu.VMEM_SHARED((QROWS, 2 * EMB), jnp.float32),
        pltpu.SemaphoreType.DMA,
        pltpu.SemaphoreType.DMA,
        pltpu.SemaphoreType.DMA,
        pltpu.SemaphoreType.DMA,
    ],
)
def _sc_scatter_add(u_hbm, di_hbm, out_hbm,
                    idx_v, idxq_v, u_v, zed_v, acc_sh,
                    lsem0, lsem1, ssem0, ssem1):
    c = lax.axis_index("c")
    s = lax.axis_index("s")
    zrow = jnp.zeros((16,), jnp.float32)

    def fzero(i, carry):
        for k in range(8):
            zed_v[i, pl.ds(16 * k, 16)] = zrow
        return carry
    lax.fori_loop(0, ZROWS, fzero, 0)

    for p in range(NSLICE // 2):
        q = c * (NSLICE // 2) + p
        for j in range(SUBROWS // ZROWS):
            pltpu.sync_copy(zed_v,
                            acc_sh.at[pl.ds(s * SUBROWS + j * ZROWS, ZROWS), :])
        plsc.subcore_barrier()

        lsems = (lsem0, lsem1)
        ssems = (ssem0, ssem1)

        def step(i, carry):
            for k in range(2):
                bid = (2 * i + k) * 16 + s

                @pl.when(bid < NBLK)
                def _():
                    off = bid * EB
                    pltpu.async_copy(di_hbm.at[pl.ds(off, EB)],
                                     idx_v.at[k], lsems[k])
                    pltpu.async_copy(u_hbm.at[pl.ds(off, EB), :],
                                     u_v.at[k], lsems[k])
            for k in range(2):
                bid = (2 * i + k) * 16 + s

                @pl.when(bid < NBLK)
                def _():
                    off = bid * EB
                    pltpu.make_async_copy(di_hbm.at[pl.ds(off, EB)],
                                          idx_v.at[k], lsems[k]).wait()
                    pltpu.make_async_copy(u_hbm.at[pl.ds(off, EB), :],
                                          u_v.at[k], lsems[k]).wait()
                    junk = N_Q + s
                    for kk in range(EB // 16):
                        iv = idx_v[k, pl.ds(16 * kk, 16)] - q * N_Q
                        ok = (iv >= 0) & (iv < N_Q)
                        idxq_v[k, pl.ds(16 * kk, 16)] = jnp.where(ok, iv, junk)
                    pltpu.async_copy(u_v.at[k], acc_sh.at[idxq_v.at[k]],
                                     ssems[k], add=True)
            for k in range(2):
                bid = (2 * i + k) * 16 + s

                @pl.when(bid < NBLK)
                def _():
                    pltpu.make_async_copy(u_v.at[k], acc_sh.at[idxq_v.at[k]],
                                          ssems[k]).wait()
            return carry
        lax.fori_loop(0, (NBLK + 31) // 32, step, 0)
        plsc.subcore_barrier()

        rows = pl.ds(s * SUBROWS, SUBROWS)
        out_rows = pl.ds(q * QROWS + s * SUBROWS, SUBROWS)
        pltpu.sync_copy(acc_sh.at[rows, :], out_hbm.at[out_rows, :])
        plsc.subcore_barrier()


def _ln_rows(x, g, b):
    m = jnp.mean(x, axis=-1, keepdims=True)
    v = jnp.mean((x - m) * (x - m), axis=-1, keepdims=True)
    return (x - m) / jnp.sqrt(v + EPS) * g + b


def _tc_embed_body(cons_ref, var_ref,
                   clng_ref, clnb_ref, cw1_ref, cb1_ref, cw2_ref, cb2_ref,
                   vlng_ref, vlnb_ref, vw1_ref, vb1_ref, vw2_ref, vb2_ref,
                   wl1_ref, ab1_ref, wr1_ref, wl2_ref, ab2_ref,
                   consemb_ref, varemb_ref, t1_ref, a2_ref):
    cons = _ln_rows(cons_ref[...], clng_ref[...], clnb_ref[...])
    cons = jnp.maximum(jnp.dot(cons, cw1_ref[...],
                               preferred_element_type=jnp.float32)
                       + cb1_ref[...], 0.0)
    cons = jnp.maximum(jnp.dot(cons, cw2_ref[...],
                               preferred_element_type=jnp.float32)
                       + cb2_ref[...], 0.0)
    consemb_ref[...] = cons
    a1 = jnp.dot(cons, wl1_ref[...],
                 preferred_element_type=jnp.float32) + ab1_ref[...]

    var = _ln_rows(var_ref[...], vlng_ref[...], vlnb_ref[...])
    var = jnp.maximum(jnp.dot(var, vw1_ref[...],
                              preferred_element_type=jnp.float32)
                      + vb1_ref[...], 0.0)
    var = jnp.maximum(jnp.dot(var, vw2_ref[...],
                              preferred_element_type=jnp.float32)
                      + vb2_ref[...], 0.0)
    varemb_ref[...] = var
    b1 = jnp.dot(var, wr1_ref[...],
                 preferred_element_type=jnp.float32)
    t1_ref[...] = jnp.concatenate([a1, b1], axis=-1)
    a2_ref[...] = jnp.dot(var, wl2_ref[...],
                          preferred_element_type=jnp.float32) + ab2_ref[...]


def _tc_edge_ln_body(tab_ref, g_ref, b_ref, wf_ref, bf_ref, zero_ref, u_ref):
    tab = tab_ref[...]
    t = tab[:, :EMB] + tab[:, EMB:]
    u = jnp.maximum(_ln_rows(t, g_ref[...], b_ref[...]), 0.0)
    msg = jnp.dot(u, wf_ref[...], preferred_element_type=jnp.float32) + bf_ref[...]
    pad = jnp.broadcast_to(zero_ref[...], msg.shape)
    u_ref[...] = jnp.concatenate([msg, pad], axis=-1)


def _tc_post_node(ag_ref, right_ref,
                  pcg_ref, pcb_ref,
                  wo1_ref, bo1_ref, wo2_ref, bo2_ref):
    h1 = _ln_rows(ag_ref[...][:, :EMB], pcg_ref[...], pcb_ref[...])
    cat = jnp.concatenate([h1, right_ref[...]], axis=-1)
    h = jnp.maximum(jnp.dot(cat, wo1_ref[...],
                            preferred_element_type=jnp.float32)
                    + bo1_ref[...], 0.0)
    return jnp.dot(h, wo2_ref[...],
                   preferred_element_type=jnp.float32) + bo2_ref[...]


def _tc_post1_body(ag_ref, right_ref,
                   pcg_ref, pcb_ref,
                   wo1_ref, bo1_ref, wo2_ref, bo2_ref,
                   a2_ref, wr2_ref, t2_ref):
    node = _tc_post_node(ag_ref, right_ref,
                         pcg_ref, pcb_ref,
                         wo1_ref, bo1_ref, wo2_ref, bo2_ref)
    b2 = jnp.dot(node, wr2_ref[...], preferred_element_type=jnp.float32)
    t2_ref[...] = jnp.concatenate([a2_ref[...], b2], axis=-1)


def _tc_post2_body(ag_ref, right_ref,
                   pcg_ref, pcb_ref,
                   wo1_ref, bo1_ref, wo2_ref, bo2_ref,
                   w1_ref, b1_ref, w2_ref, out_ref):
    node = _tc_post_node(ag_ref, right_ref,
                         pcg_ref, pcb_ref,
                         wo1_ref, bo1_ref, wo2_ref, bo2_ref)
    h = jnp.maximum(jnp.dot(node, w1_ref[...],
                            preferred_element_type=jnp.float32)
                    + b1_ref[...], 0.0)
    out_ref[...] = jnp.dot(h, w2_ref[...],
                           preferred_element_type=jnp.float32)


def _full_spec(shape):
    return pl.BlockSpec(shape, lambda i: (0,) * len(shape))


def _row_spec(blk, shape):
    return pl.BlockSpec((blk,) + shape[1:],
                        lambda i: (i,) + (0,) * (len(shape) - 1))


def _tc_call(body, grid, n_rows, blk, ins, row_mask, out_shapes):
    in_specs = [_row_spec(blk, x.shape) if is_row else _full_spec(x.shape)
                for x, is_row in zip(ins, row_mask)]
    out_specs = [_row_spec(blk, s) for s in out_shapes]
    return pl.pallas_call(
        body,
        grid=(grid,),
        in_specs=in_specs,
        out_specs=out_specs,
        out_shape=[jax.ShapeDtypeStruct(s, jnp.float32) for s in out_shapes],
    )(*ins)


def _conv_edge_phase(t, ai, bi, fg, fb, wf, bf):
    tab = _sc_gather(t, ai, bi)
    blk = 4000
    zero = jnp.zeros((1, EMB), jnp.float32)
    return _tc_call(
        _tc_edge_ln_body, N_EDGES // blk, N_EDGES, blk,
        [tab, fg, fb, wf, bf, zero], [True] + [False] * 5,
        [(N_EDGES, 2 * EMB)])[0]


def _r2(x):
    return x.reshape(1, -1)


def kernel(constraint_features, edge_indices, edge_features, variable_features, params):
    del edge_features
    p = params
    cidx = edge_indices[0]
    vidx = edge_indices[1]
    vc, cv = p['vc'], p['cv']
    econst = p['edge_ln_b'][0]
    ab1 = _r2(vc['bl'] + econst * vc['We'][0])
    ab2 = _r2(cv['bl'] + econst * cv['We'][0])

    blk_n = 2000
    grid_n = N_NODES // blk_n
    consemb, varemb, t1, a2 = _tc_call(
        _tc_embed_body, grid_n, N_NODES, blk_n,
        [constraint_features, variable_features,
         _r2(p['cons_ln_g']), _r2(p['cons_ln_b']),
         p['cons_W1'], _r2(p['cons_b1']), p['cons_W2'], _r2(p['cons_b2']),
         _r2(p['var_ln_g']), _r2(p['var_ln_b']),
         p['var_W1'], _r2(p['var_b1']), p['var_W2'], _r2(p['var_b2']),
         vc['Wl'], ab1, vc['Wr'], cv['Wl'], ab2],
        [True, True] + [False] * 17,
        [(N_NODES, EMB), (N_NODES, EMB), (N_NODES, 2 * EMB), (N_NODES, EMB)])

    u = _conv_edge_phase(t1, cidx, vidx,
                         _r2(vc['fln_g']), _r2(vc['fln_b']),
                         vc['Wf'], _r2(vc['bf']))
    ag = _assemble_agg(_sc_scatter_add(u, cidx))
    t2 = _tc_call(
        _tc_post1_body, grid_n, N_NODES, blk_n,
        [ag, consemb,
         _r2(vc['pc_g']), _r2(vc['pc_b']),
         vc['Wo1'], _r2(vc['bo1']), vc['Wo2'], _r2(vc['bo2']),
         a2, cv['Wr']],
        [True, True] + [False] * 6 + [True, False],
        [(N_NODES, 2 * EMB)])[0]

    u = _conv_edge_phase(t2, vidx, cidx,
                         _r2(cv['fln_g']), _r2(cv['fln_b']),
                         cv['Wf'], _r2(cv['bf']))
    ag = _assemble_agg(_sc_scatter_add(u, vidx))
    out = _tc_call(
        _tc_post2_body, grid_n, N_NODES, blk_n,
        [ag, varemb,
         _r2(cv['pc_g']), _r2(cv['pc_b']),
         cv['Wo1'], _r2(cv['bo1']), cv['Wo2'], _r2(cv['bo2']),
         p['out_W1'], _r2(p['out_b1']), p['out_W2']],
        [True, True] + [False] * 9,
        [(N_NODES, 1)])[0]
    return jnp.squeeze(out, -1)


def _assemble_agg(res):
    q = res.reshape(NSLICE, QROWS, 2 * EMB)[:, :N_Q, :]
    return q.reshape(NSLICE * N_Q, 2 * EMB)[:N_NODES]
